# Initial kernel scaffold; baseline (speedup 1.0000x reference)
import jax
import jax.numpy as jnp
from jax.experimental import pallas as pl


def kernel(input_tensor, table):
    raise NotImplementedError("write your pallas kernel here")


# speedup vs baseline: 4.0895x; 4.0895x over previous
import functools

import jax
import jax.numpy as jnp
from jax import lax
from jax.experimental import pallas as pl
from jax.experimental.pallas import tpu as pltpu
from jax.experimental.pallas import tpu_sc as plsc

EMBED_DIM = 64
NC = 2
NS = 16
NW = NC * NS
G = 128


@functools.lru_cache(maxsize=None)
def _make_kernel(B):
    assert B % (NW * G) == 0
    b_per_w = B // NW
    ng = b_per_w // G
    mesh = plsc.VectorSubcoreMesh(core_axis_name="c", subcore_axis_name="s")

    @functools.partial(
        pl.kernel,
        mesh=mesh,
        out_type=jax.ShapeDtypeStruct((B, EMBED_DIM), jnp.float32),
        scratch_types=[
            pltpu.VMEM((ng, G), jnp.int32),
            pltpu.VMEM((G, EMBED_DIM), jnp.float32),
            pltpu.SemaphoreType.DMA,
        ],
        compiler_params=pltpu.CompilerParams(use_tc_tiling_on_sc=False),
    )
    def k(idx_hbm, table_hbm, out_hbm, idx_v, rows_v, sem):
        wid = lax.axis_index("s") * NC + lax.axis_index("c")
        base = wid * b_per_w
        pltpu.sync_copy(idx_hbm.at[wid], idx_v)

        def body(j, carry):
            pltpu.async_copy(table_hbm.at[idx_v.at[j]], rows_v, sem).wait()
            pltpu.sync_copy(rows_v, out_hbm.at[pl.ds(base + j * G, G)])
            return carry

        lax.fori_loop(0, ng, body, 0)

    return k


def kernel(input_tensor, table):
    batch, hist = input_tensor.shape
    b_total = batch * hist
    idx = input_tensor.astype(jnp.int32).reshape(NW, b_total // (NW * G), G)
    out = _make_kernel(b_total)(idx, table)
    return out.reshape(batch, hist, EMBED_DIM)


# speedup vs baseline: 4.6701x; 1.1420x over previous
import functools

import jax
import jax.numpy as jnp
from jax import lax
from jax.experimental import pallas as pl
from jax.experimental.pallas import tpu as pltpu
from jax.experimental.pallas import tpu_sc as plsc

EMBED_DIM = 64
NC = 2
NS = 16
NW = NC * NS
G = 128
NBUF = 5


@functools.lru_cache(maxsize=None)
def _make_kernel(B):
    assert B % (NW * G) == 0
    b_per_w = B // NW
    ng = b_per_w // G
    assert ng % NBUF == 0
    n_outer = ng // NBUF
    mesh = plsc.VectorSubcoreMesh(core_axis_name="c", subcore_axis_name="s")

    scratch = [pltpu.VMEM((ng, G), jnp.int32)]
    scratch += [pltpu.VMEM((G, EMBED_DIM), jnp.float32) for _ in range(NBUF)]
    scratch += [pltpu.SemaphoreType.DMA for _ in range(2 * NBUF)]

    @functools.partial(
        pl.kernel,
        mesh=mesh,
        out_type=jax.ShapeDtypeStruct((B, EMBED_DIM), jnp.float32),
        scratch_types=scratch,
        compiler_params=pltpu.CompilerParams(use_tc_tiling_on_sc=False),
    )
    def k(idx_hbm, table_hbm, out_hbm, idx_v, *rest):
        bufs = rest[:NBUF]
        gsem = rest[NBUF:2 * NBUF]
        osem = rest[2 * NBUF:]
        wid = lax.axis_index("s") * NC + lax.axis_index("c")
        base = wid * b_per_w
        pltpu.sync_copy(idx_hbm.at[wid], idx_v)

        for b in range(NBUF):
            pltpu.async_copy(table_hbm.at[idx_v.at[b]], bufs[b], gsem[b])

        def step(j, b, j_next):
            pltpu.make_async_copy(
                table_hbm.at[idx_v.at[j]], bufs[b], gsem[b]).wait()
            out_cp = pltpu.async_copy(
                bufs[b], out_hbm.at[pl.ds(base + j * G, G)], osem[b])
            out_cp.wait()
            if j_next is not None:
                pltpu.async_copy(
                    table_hbm.at[idx_v.at[j_next]], bufs[b], gsem[b])

        def outer(t, carry):
            for b in range(NBUF):
                j = t * NBUF + b
                step(j, b, j + NBUF)
            return carry

        lax.fori_loop(0, n_outer - 1, outer, 0)
        for b in range(NBUF):
            step((n_outer - 1) * NBUF + b, b, None)

    return k


def kernel(input_tensor, table):
    batch, hist = input_tensor.shape
    b_total = batch * hist
    idx = input_tensor.astype(jnp.int32).reshape(NW, b_total // (NW * G), G)
    out = _make_kernel(b_total)(idx, table)
    return out.reshape(batch, hist, EMBED_DIM)


# speedup vs baseline: 7.5070x; 1.6075x over previous
import functools

import jax
import jax.numpy as jnp
from jax import lax
from jax.experimental import pallas as pl
from jax.experimental.pallas import tpu as pltpu
from jax.experimental.pallas import tpu_sc as plsc

EMB = 64
NC = 2
NS = 16
NW = NC * NS
LANES = 16
UNROLL = 8


@functools.lru_cache(maxsize=None)
def _make_kernel(hist, batch, vocab):
    n_rounds = EMB // NW
    groups = batch // LANES
    mesh = plsc.VectorSubcoreMesh(core_axis_name="c", subcore_axis_name="s")

    scratch = [
        pltpu.VMEM((vocab,), jnp.float32),
        pltpu.VMEM((batch,), jnp.int32),
        pltpu.VMEM((batch,), jnp.int32),
        pltpu.VMEM((batch,), jnp.float32),
        pltpu.VMEM((batch,), jnp.float32),
        pltpu.SemaphoreType.DMA,
        pltpu.SemaphoreType.DMA,
        pltpu.SemaphoreType.DMA,
        pltpu.SemaphoreType.DMA,
    ]

    @functools.partial(
        pl.kernel,
        mesh=mesh,
        out_type=jax.ShapeDtypeStruct((hist, EMB, batch), jnp.float32),
        scratch_types=scratch,
        compiler_params=pltpu.CompilerParams(
            use_tc_tiling_on_sc=True, needs_layout_passes=False),
    )
    def k(idx_hbm, table_hbm, out_hbm, row_v, ix0, ix1, ov0, ov1,
          isem0, isem1, osem0, osem1):
        ix = (ix0, ix1)
        ov = (ov0, ov1)
        isem = (isem0, isem1)
        osem = (osem0, osem1)
        wid = lax.axis_index("s") * NC + lax.axis_index("c")

        def drain_out(b):
            pltpu.make_async_copy(ov[b], out_hbm.at[0, 0], osem[b]).wait()

        def gather_h(ixb, ovb):
            def body(g, carry):
                for u in range(UNROLL):
                    off = (g * UNROLL + u) * LANES
                    iv = ixb[pl.ds(off, LANES)]
                    ovb[pl.ds(off, LANES)] = plsc.load_gather(row_v, [iv])
                return carry
            lax.fori_loop(0, groups // UNROLL, body, 0)

        for r in range(n_rounds):
            e = wid + NW * r
            pltpu.sync_copy(table_hbm.at[e], row_v)
            for b in range(2):
                pltpu.async_copy(idx_hbm.at[b], ix[b], isem[b])

            def h_pair(t, carry):
                for b in range(2):
                    h = t * 2 + b
                    pltpu.make_async_copy(
                        idx_hbm.at[h], ix[b], isem[b]).wait()
                    if r == 0:
                        @pl.when(h >= 2)
                        def _():
                            drain_out(b)
                    else:
                        drain_out(b)
                    gather_h(ix[b], ov[b])
                    pltpu.async_copy(ov[b], out_hbm.at[h, e], osem[b])

                    @pl.when(h + 2 < hist)
                    def _():
                        pltpu.async_copy(idx_hbm.at[h + 2], ix[b], isem[b])
                return carry

            lax.fori_loop(0, hist // 2, h_pair, 0)

        for b in range(2):
            drain_out(b)

    return k


def kernel(input_tensor, table):
    batch, hist = input_tensor.shape
    vocab, emb = table.shape
    idx_t = input_tensor.T.astype(jnp.int32)
    table_t = table.T
    out_t = _make_kernel(hist, batch, vocab)(idx_t, table_t)
    return jnp.transpose(out_t, (2, 0, 1))


# speedup vs baseline: 7.5561x; 1.0065x over previous
import functools

import jax
import jax.numpy as jnp
from jax import lax
from jax.experimental import pallas as pl
from jax.experimental.pallas import tpu as pltpu
from jax.experimental.pallas import tpu_sc as plsc

EMB = 64
NC = 2
NS = 16
NW = NC * NS
LANES = 16
UNROLL = 16


@functools.lru_cache(maxsize=None)
def _make_kernel(hist, batch, vocab):
    n_rounds = EMB // NW
    groups = batch // LANES
    mesh = plsc.VectorSubcoreMesh(core_axis_name="c", subcore_axis_name="s")

    scratch = [
        pltpu.VMEM((vocab,), jnp.float32),
        pltpu.VMEM((batch,), jnp.int32),
        pltpu.VMEM((batch,), jnp.int32),
        pltpu.VMEM((batch,), jnp.float32),
        pltpu.VMEM((batch,), jnp.float32),
        pltpu.SemaphoreType.DMA,
        pltpu.SemaphoreType.DMA,
        pltpu.SemaphoreType.DMA,
        pltpu.SemaphoreType.DMA,
    ]

    @functools.partial(
        pl.kernel,
        mesh=mesh,
        out_type=jax.ShapeDtypeStruct((hist, EMB, batch), jnp.float32),
        scratch_types=scratch,
        compiler_params=pltpu.CompilerParams(
            use_tc_tiling_on_sc=True, needs_layout_passes=False),
    )
    def k(idx_hbm, table_hbm, out_hbm, row_v, ix0, ix1, ov0, ov1,
          isem0, isem1, osem0, osem1):
        ix = (ix0, ix1)
        ov = (ov0, ov1)
        isem = (isem0, isem1)
        osem = (osem0, osem1)
        wid = lax.axis_index("s") * NC + lax.axis_index("c")

        def drain_out(b):
            pltpu.make_async_copy(ov[b], out_hbm.at[0, 0], osem[b]).wait()

        def gather_h(ixb, ovb):
            def body(g, carry):
                for u in range(UNROLL):
                    off = (g * UNROLL + u) * LANES
                    iv = ixb[pl.ds(off, LANES)]
                    ovb[pl.ds(off, LANES)] = plsc.load_gather(row_v, [iv])
                return carry
            lax.fori_loop(0, groups // UNROLL, body, 0)

        for r in range(n_rounds):
            e = wid + NW * r
            pltpu.sync_copy(table_hbm.at[e], row_v)
            for b in range(2):
                pltpu.async_copy(idx_hbm.at[b], ix[b], isem[b])

            def h_pair(t, carry):
                for b in range(2):
                    h = t * 2 + b
                    pltpu.make_async_copy(
                        idx_hbm.at[h], ix[b], isem[b]).wait()
                    if r == 0:
                        @pl.when(h >= 2)
                        def _():
                            drain_out(b)
                    else:
                        drain_out(b)
                    gather_h(ix[b], ov[b])
                    pltpu.async_copy(ov[b], out_hbm.at[h, e], osem[b])

                    @pl.when(h + 2 < hist)
                    def _():
                        pltpu.async_copy(idx_hbm.at[h + 2], ix[b], isem[b])
                return carry

            lax.fori_loop(0, hist // 2, h_pair, 0)

        for b in range(2):
            drain_out(b)

    return k


def kernel(input_tensor, table):
    batch, hist = input_tensor.shape
    vocab, emb = table.shape
    idx_t = input_tensor.T.astype(jnp.int32)
    table_t = table.T
    out_t = _make_kernel(hist, batch, vocab)(idx_t, table_t)
    return jnp.transpose(out_t, (2, 0, 1))


# speedup vs baseline: 9.9994x; 1.3234x over previous
import functools

import jax
import jax.numpy as jnp
from jax import lax
from jax.experimental import pallas as pl
from jax.experimental.pallas import tpu as pltpu
from jax.experimental.pallas import tpu_sc as plsc

EMB = 64
NC = 2
NS = 16
NW = NC * NS
LANES = 16
UNROLL = 16


@functools.lru_cache(maxsize=None)
def _make_kernel(hist, batch, vocab):
    n_rounds = EMB // NW
    groups = batch // LANES
    mesh = plsc.VectorSubcoreMesh(core_axis_name="c", subcore_axis_name="s")

    scratch = [
        pltpu.VMEM((vocab,), jnp.float32),
        pltpu.VMEM((batch,), jnp.int32),
        pltpu.VMEM((batch,), jnp.int32),
        pltpu.VMEM((batch,), jnp.float32),
        pltpu.VMEM((batch,), jnp.float32),
        pltpu.SemaphoreType.DMA,
        pltpu.SemaphoreType.DMA,
        pltpu.SemaphoreType.DMA,
        pltpu.SemaphoreType.DMA,
    ]

    @functools.partial(
        pl.kernel,
        mesh=mesh,
        out_type=jax.ShapeDtypeStruct((hist, EMB, batch), jnp.float32),
        scratch_types=scratch,
        compiler_params=pltpu.CompilerParams(
            use_tc_tiling_on_sc=True, needs_layout_passes=False),
    )
    def k(idx_hbm, table_hbm, out_hbm, row_v, ix0, ix1, ov0, ov1,
          isem0, isem1, osem0, osem1):
        ix = (ix0, ix1)
        ov = (ov0, ov1)
        isem = (isem0, isem1)
        osem = (osem0, osem1)
        wid = lax.axis_index("s") * NC + lax.axis_index("c")

        def drain_out(b):
            pltpu.make_async_copy(ov[b], out_hbm.at[0, 0], osem[b]).wait()

        def gather_h(ixb, ovb):
            @plsc.parallel_loop(0, batch, LANES, unroll=UNROLL)
            def _(off):
                iv = ixb[pl.ds(off, LANES)]
                ovb[pl.ds(off, LANES)] = plsc.load_gather(row_v, [iv])

        for r in range(n_rounds):
            e = wid + NW * r
            pltpu.sync_copy(table_hbm.at[e], row_v)
            for b in range(2):
                pltpu.async_copy(idx_hbm.at[b], ix[b], isem[b])

            def h_pair(t, carry):
                for b in range(2):
                    h = t * 2 + b
                    pltpu.make_async_copy(
                        idx_hbm.at[h], ix[b], isem[b]).wait()
                    if r == 0:
                        @pl.when(h >= 2)
                        def _():
                            drain_out(b)
                    else:
                        drain_out(b)
                    gather_h(ix[b], ov[b])
                    pltpu.async_copy(ov[b], out_hbm.at[h, e], osem[b])

                    @pl.when(h + 2 < hist)
                    def _():
                        pltpu.async_copy(idx_hbm.at[h + 2], ix[b], isem[b])
                return carry

            lax.fori_loop(0, hist // 2, h_pair, 0)

        for b in range(2):
            drain_out(b)

    return k


def kernel(input_tensor, table):
    batch, hist = input_tensor.shape
    vocab, emb = table.shape
    idx_t = input_tensor.T.astype(jnp.int32)
    table_t = table.T
    out_t = _make_kernel(hist, batch, vocab)(idx_t, table_t)
    return jnp.transpose(out_t, (2, 0, 1))
